# Initial kernel scaffold; baseline (speedup 1.0000x reference)
import jax
import jax.numpy as jnp
from jax.experimental import pallas as pl


def kernel(token_ids, embedding_table):
    raise NotImplementedError("write your pallas kernel here")


# speedup vs baseline: 4.8073x; 4.8073x over previous
import functools

import jax
import jax.numpy as jnp
from jax import lax
from jax.experimental import pallas as pl
from jax.experimental.pallas import tpu as pltpu
from jax.experimental.pallas import tpu_sc as plsc

DIM = 32

_info = plsc.get_sparse_core_info()
_NC, _NS = _info.num_cores, _info.num_subcores
NW = _NC * _NS


def _make_gather(B: int, chunk: int):
    b_per_w = B // NW
    n_chunks = b_per_w // chunk
    mesh = plsc.VectorSubcoreMesh(core_axis_name="c", subcore_axis_name="s")

    @functools.partial(
        pl.kernel,
        mesh=mesh,
        out_type=jax.ShapeDtypeStruct((B, DIM), jnp.float32),
        scratch_types=[
            pltpu.VMEM((chunk,), jnp.int32),
            pltpu.VMEM((chunk, DIM), jnp.float32),
            pltpu.SemaphoreType.DMA,
        ],
        compiler_params=pltpu.CompilerParams(use_tc_tiling_on_sc=False),
    )
    def gather_kernel(idx_hbm, table_hbm, out_hbm, idx_v, rows_v, sem):
        wid = lax.axis_index("s") * _NC + lax.axis_index("c")
        base = wid * b_per_w

        def body(g, carry):
            off = base + g * chunk
            pltpu.sync_copy(idx_hbm.at[pl.ds(off, chunk)], idx_v)
            pltpu.async_copy(table_hbm.at[idx_v], rows_v, sem).wait()
            pltpu.sync_copy(rows_v, out_hbm.at[pl.ds(off, chunk)])
            return carry

        lax.fori_loop(0, n_chunks, body, 0)

    return gather_kernel


def kernel(token_ids, embedding_table):
    B = token_ids.shape[0] * token_ids.shape[1]
    idx = token_ids.reshape(B).astype(jnp.int32)
    out = _make_gather(B, 1024)(idx, embedding_table)
    return out.reshape(token_ids.shape + (DIM,))


# speedup vs baseline: 5.0499x; 1.0505x over previous
import functools

import jax
import jax.numpy as jnp
from jax import lax
from jax.experimental import pallas as pl
from jax.experimental.pallas import tpu as pltpu
from jax.experimental.pallas import tpu_sc as plsc

DIM = 32

_info = plsc.get_sparse_core_info()
_NC, _NS = _info.num_cores, _info.num_subcores
NW = _NC * _NS


def _make_gather(B: int, chunk: int, nbuf: int):
    b_per_w = B // NW
    n_chunks = b_per_w // chunk
    n_outer = n_chunks // nbuf
    assert b_per_w % chunk == 0 and n_chunks % nbuf == 0 and n_outer >= 3
    mesh = plsc.VectorSubcoreMesh(core_axis_name="c", subcore_axis_name="s")

    @functools.partial(
        pl.kernel,
        mesh=mesh,
        out_type=jax.ShapeDtypeStruct((B, DIM), jnp.float32),
        scratch_types=[
            pltpu.VMEM((nbuf, chunk), jnp.int32),
            pltpu.VMEM((nbuf, chunk, DIM), jnp.float32),
        ]
        + [pltpu.SemaphoreType.DMA] * (3 * nbuf),
        compiler_params=pltpu.CompilerParams(use_tc_tiling_on_sc=False),
    )
    def gather_kernel(idx_hbm, table_hbm, out_hbm, idx_v, rows_v, *sems):
        isem = sems[0:nbuf]
        gsem = sems[nbuf : 2 * nbuf]
        osem = sems[2 * nbuf : 3 * nbuf]
        wid = lax.axis_index("s") * _NC + lax.axis_index("c")
        base = wid * b_per_w

        def idx_start(c, b):
            pltpu.async_copy(
                idx_hbm.at[pl.ds(base + c * chunk, chunk)], idx_v.at[b], isem[b]
            )

        def idx_wait(b):
            pltpu.make_async_copy(
                idx_hbm.at[pl.ds(0, chunk)], idx_v.at[b], isem[b]
            ).wait()

        def gather_start(b):
            pltpu.async_copy(table_hbm.at[idx_v.at[b]], rows_v.at[b], gsem[b])

        def gather_wait(b):
            pltpu.make_async_copy(
                table_hbm.at[idx_v.at[b]], rows_v.at[b], gsem[b]
            ).wait()

        def out_start(c, b):
            pltpu.async_copy(
                rows_v.at[b], out_hbm.at[pl.ds(base + c * chunk, chunk)], osem[b]
            )

        def out_wait(b):
            pltpu.make_async_copy(
                rows_v.at[b], out_hbm.at[pl.ds(0, chunk)], osem[b]
            ).wait()

        for b in range(nbuf):
            idx_start(b, b)
        idx_wait(0)
        gather_start(0)

        for b in range(1, nbuf):
            idx_wait(b)
            gather_start(b)
            gather_wait(b - 1)
            out_start(b - 1, b - 1)
            idx_start(b - 1 + nbuf, b - 1)

        def outer(o, carry):
            for b in range(nbuf):
                g = o * nbuf + b
                bp = (b - 1) % nbuf
                idx_wait(b)
                out_wait(b)
                gather_start(b)
                gather_wait(bp)
                out_start(g - 1, bp)
                idx_start(g - 1 + nbuf, bp)
            return carry

        lax.fori_loop(1, n_outer - 1, outer, 0)

        for b in range(nbuf):
            g = (n_outer - 1) * nbuf + b
            bp = (b - 1) % nbuf
            idx_wait(b)
            out_wait(b)
            gather_start(b)
            gather_wait(bp)
            out_start(g - 1, bp)
            if b == 0:
                idx_start(g - 1 + nbuf, bp)

        blast = nbuf - 1
        gather_wait(blast)
        out_start(n_chunks - 1, blast)
        for b in range(nbuf):
            out_wait(b)

    return gather_kernel


def kernel(token_ids, embedding_table):
    B = token_ids.shape[0] * token_ids.shape[1]
    idx = token_ids.reshape(B).astype(jnp.int32)
    out = _make_gather(B, 1600, 2)(idx, embedding_table)
    return out.reshape(token_ids.shape + (DIM,))


# speedup vs baseline: 5.0518x; 1.0004x over previous
import functools

import jax
import jax.numpy as jnp
from jax import lax
from jax.experimental import pallas as pl
from jax.experimental.pallas import tpu as pltpu
from jax.experimental.pallas import tpu_sc as plsc

DIM = 32

_info = plsc.get_sparse_core_info()
_NC, _NS = _info.num_cores, _info.num_subcores
NW = _NC * _NS


def _make_gather(B: int, chunk: int, nbuf: int):
    b_per_w = B // NW
    n_chunks = b_per_w // chunk
    n_outer = n_chunks // nbuf
    assert b_per_w % chunk == 0 and n_chunks % nbuf == 0 and n_outer >= 3
    mesh = plsc.VectorSubcoreMesh(core_axis_name="c", subcore_axis_name="s")

    @functools.partial(
        pl.kernel,
        mesh=mesh,
        out_type=jax.ShapeDtypeStruct((B, DIM), jnp.float32),
        scratch_types=[
            pltpu.VMEM((nbuf, chunk), jnp.int32),
            pltpu.VMEM((nbuf, chunk, DIM), jnp.float32),
        ]
        + [pltpu.SemaphoreType.DMA] * (3 * nbuf),
        compiler_params=pltpu.CompilerParams(use_tc_tiling_on_sc=False),
    )
    def gather_kernel(idx_hbm, table_hbm, out_hbm, idx_v, rows_v, *sems):
        isem = sems[0:nbuf]
        gsem = sems[nbuf : 2 * nbuf]
        osem = sems[2 * nbuf : 3 * nbuf]
        wid = lax.axis_index("s") * _NC + lax.axis_index("c")
        base = wid * b_per_w

        def idx_start(c, b):
            pltpu.async_copy(
                idx_hbm.at[pl.ds(base + c * chunk, chunk)], idx_v.at[b], isem[b]
            )

        def idx_wait(b):
            pltpu.make_async_copy(
                idx_hbm.at[pl.ds(0, chunk)], idx_v.at[b], isem[b]
            ).wait()

        def gather_start(b):
            pltpu.async_copy(table_hbm.at[idx_v.at[b]], rows_v.at[b], gsem[b])

        def gather_wait(b):
            pltpu.make_async_copy(
                table_hbm.at[idx_v.at[b]], rows_v.at[b], gsem[b]
            ).wait()

        def out_start(c, b):
            pltpu.async_copy(
                rows_v.at[b], out_hbm.at[pl.ds(base + c * chunk, chunk)], osem[b]
            )

        def out_wait(b):
            pltpu.make_async_copy(
                rows_v.at[b], out_hbm.at[pl.ds(0, chunk)], osem[b]
            ).wait()

        for b in range(nbuf):
            idx_start(b, b)
        idx_wait(0)
        gather_start(0)

        for b in range(1, nbuf):
            idx_wait(b)
            gather_start(b)
            gather_wait(b - 1)
            out_start(b - 1, b - 1)
            idx_start(b - 1 + nbuf, b - 1)

        def outer(o, carry):
            for b in range(nbuf):
                g = o * nbuf + b
                bp = (b - 1) % nbuf
                idx_wait(b)
                out_wait(b)
                gather_start(b)
                gather_wait(bp)
                out_start(g - 1, bp)
                idx_start(g - 1 + nbuf, bp)
            return carry

        lax.fori_loop(1, n_outer - 1, outer, 0)

        for b in range(nbuf):
            g = (n_outer - 1) * nbuf + b
            bp = (b - 1) % nbuf
            idx_wait(b)
            out_wait(b)
            gather_start(b)
            gather_wait(bp)
            out_start(g - 1, bp)
            if b == 0:
                idx_start(g - 1 + nbuf, bp)

        blast = nbuf - 1
        gather_wait(blast)
        out_start(n_chunks - 1, blast)
        for b in range(nbuf):
            out_wait(b)

    return gather_kernel


def kernel(token_ids, embedding_table):
    B = token_ids.shape[0] * token_ids.shape[1]
    idx = token_ids.reshape(B).astype(jnp.int32)
    out = _make_gather(B, 800, 4)(idx, embedding_table)
    return out.reshape(token_ids.shape + (DIM,))


# speedup vs baseline: 5.5418x; 1.0970x over previous
import functools

import jax
import jax.numpy as jnp
from jax import lax
from jax.experimental import pallas as pl
from jax.experimental.pallas import tpu as pltpu
from jax.experimental.pallas import tpu_sc as plsc

DIM = 32

_info = plsc.get_sparse_core_info()
_NC, _NS = _info.num_cores, _info.num_subcores
NW = _NC * _NS


def _make_gather(B: int, chunk: int, nbuf: int):
    b_per_w = B // NW
    n_chunks = b_per_w // chunk
    n_outer = n_chunks // nbuf
    assert b_per_w % chunk == 0 and n_chunks % nbuf == 0 and n_outer >= 3
    mesh = plsc.VectorSubcoreMesh(core_axis_name="c", subcore_axis_name="s")

    @functools.partial(
        pl.kernel,
        mesh=mesh,
        out_type=jax.ShapeDtypeStruct((B, DIM), jnp.float32),
        scratch_types=[
            pltpu.VMEM((nbuf, chunk), jnp.int32),
            pltpu.VMEM((nbuf, chunk, DIM), jnp.float32),
        ]
        + [pltpu.SemaphoreType.DMA] * (3 * nbuf),
        compiler_params=pltpu.CompilerParams(use_tc_tiling_on_sc=False),
    )
    def gather_kernel(idx_hbm, table_hbm, out_hbm, idx_v, rows_v, *sems):
        isem = sems[0:nbuf]
        gsem = sems[nbuf : 2 * nbuf]
        osem = sems[2 * nbuf : 3 * nbuf]
        wid = lax.axis_index("s") * _NC + lax.axis_index("c")
        base = wid * b_per_w

        def idx_start(c, b):
            pltpu.async_copy(
                idx_hbm.at[pl.ds(base + c * chunk, chunk)], idx_v.at[b], isem[b]
            )

        def idx_wait(b):
            pltpu.make_async_copy(
                idx_hbm.at[pl.ds(0, chunk)], idx_v.at[b], isem[b]
            ).wait()

        def gather_start(b):
            pltpu.async_copy(table_hbm.at[idx_v.at[b]], rows_v.at[b], gsem[b])

        def gather_wait(b):
            pltpu.make_async_copy(
                table_hbm.at[idx_v.at[b]], rows_v.at[b], gsem[b]
            ).wait()

        def out_start(c, b):
            pltpu.async_copy(
                rows_v.at[b], out_hbm.at[pl.ds(base + c * chunk, chunk)], osem[b]
            )

        def out_wait(b):
            pltpu.make_async_copy(
                rows_v.at[b], out_hbm.at[pl.ds(0, chunk)], osem[b]
            ).wait()

        for b in range(nbuf):
            idx_start(b, b)
        idx_wait(0)
        gather_start(0)

        for b in range(1, nbuf):
            idx_wait(b)
            gather_start(b)
            gather_wait(b - 1)
            out_start(b - 1, b - 1)
            idx_start(b - 1 + nbuf, b - 1)

        def outer(o, carry):
            for b in range(nbuf):
                g = o * nbuf + b
                bp = (b - 1) % nbuf
                idx_wait(b)
                out_wait(b)
                gather_start(b)
                gather_wait(bp)
                out_start(g - 1, bp)
                idx_start(g - 1 + nbuf, bp)
            return carry

        lax.fori_loop(1, n_outer - 1, outer, 0)

        for b in range(nbuf):
            g = (n_outer - 1) * nbuf + b
            bp = (b - 1) % nbuf
            idx_wait(b)
            out_wait(b)
            gather_start(b)
            gather_wait(bp)
            out_start(g - 1, bp)
            if b == 0:
                idx_start(g - 1 + nbuf, bp)

        blast = nbuf - 1
        gather_wait(blast)
        out_start(n_chunks - 1, blast)
        for b in range(nbuf):
            out_wait(b)

    return gather_kernel


def kernel(token_ids, embedding_table):
    n_seq, n_pos = token_ids.shape
    B = n_seq * n_pos
    idx = token_ids.T.reshape(B).astype(jnp.int32)
    out = _make_gather(B, 800, 4)(idx, embedding_table)
    return out.reshape(n_pos, n_seq, DIM).transpose(1, 0, 2)
